# baseline (speedup 1.0000x reference)
import jax
import jax.numpy as jnp
from jax import lax
from jax.experimental import pallas as pl
from jax.experimental.pallas import tpu as pltpu
from jax.experimental.pallas import tpu_sc as plsc

EPS_DIV = 1e-06
LANES = 16
NUM_WORKERS = 32


def _sc_recip_gather(scale_p, idx_p, g_pad):
    b_per_w = g_pad // NUM_WORKERS
    mesh = plsc.VectorSubcoreMesh(core_axis_name="c", subcore_axis_name="s")

    def body(scale_hbm, idx_hbm, out_hbm, table_v, idx_v, out_v):
        nc = lax.axis_size("c")
        wid = lax.axis_index("s") * nc + lax.axis_index("c")
        base = wid * b_per_w
        pltpu.sync_copy(scale_hbm, table_v)
        pltpu.sync_copy(idx_hbm.at[pl.ds(base, b_per_w)], idx_v)

        def step(i, carry):
            off = pl.multiple_of(i * LANES, LANES)
            iv = idx_v[pl.ds(off, LANES)]
            v = plsc.load_gather(table_v, [iv])
            out_v[pl.ds(off, LANES)] = 1.0 / (v + EPS_DIV)
            return carry

        lax.fori_loop(0, b_per_w // LANES, step, 0)
        pltpu.sync_copy(out_v, out_hbm.at[0, pl.ds(base, b_per_w)])

    fn = pl.kernel(
        body,
        out_type=jax.ShapeDtypeStruct((1, g_pad), jnp.float32),
        mesh=mesh,
        compiler_params=pltpu.CompilerParams(needs_layout_passes=False),
        scratch_types=[
            pltpu.VMEM((scale_p.shape[0],), jnp.float32),
            pltpu.VMEM((b_per_w,), jnp.int32),
            pltpu.VMEM((b_per_w,), jnp.float32),
        ],
    )
    return fn(scale_p, idx_p)


def _tc_scale_mul(x_ng, recip_row, block_rows):
    n, g = x_ng.shape

    g_pad = recip_row.shape[1]

    def body(x_ref, r_ref, o_ref):
        o_ref[...] = x_ref[...] * r_ref[...][:, :g]

    return pl.pallas_call(
        body,
        grid=(pl.cdiv(n, block_rows),),
        in_specs=[
            pl.BlockSpec((block_rows, g), lambda i: (i, 0)),
            pl.BlockSpec((1, g_pad), lambda i: (0, 0)),
        ],
        out_specs=pl.BlockSpec((block_rows, g), lambda i: (i, 0)),
        out_shape=jax.ShapeDtypeStruct((n, g), jnp.float32),
        compiler_params=pltpu.CompilerParams(vmem_limit_bytes=128 * 1024 * 1024),
    )(x_ng, recip_row)


@jax.jit
def kernel(x_ng, scale_g, idx):
    n, g = x_ng.shape
    chunk = NUM_WORKERS * LANES
    g_pad = ((g + chunk - 1) // chunk) * chunk
    idx_p = jnp.pad(idx, (0, g_pad - g))
    recip_row = _sc_recip_gather(scale_g, idx_p, g_pad)
    return _tc_scale_mul(x_ng, recip_row, block_rows=112)


# speedup vs baseline: 1.0051x; 1.0051x over previous
import jax
import jax.numpy as jnp
from jax import lax
from jax.experimental import pallas as pl
from jax.experimental.pallas import tpu as pltpu
from jax.experimental.pallas import tpu_sc as plsc

EPS_DIV = 1e-06
LANES = 16
NUM_WORKERS = 32


def _sc_recip_gather(scale_g, idx, g_pad):
    g = idx.shape[0]
    b_per_w = g_pad // NUM_WORKERS
    tail_base = (NUM_WORKERS - 1) * b_per_w
    tail_len = g - tail_base
    tail_out = ((tail_len + 127) // 128) * 128
    mesh = plsc.VectorSubcoreMesh(core_axis_name="c", subcore_axis_name="s")

    def body(scale_hbm, idx_hbm, out_hbm, table_v, idx_v, out_v, sem_t, sem_i):
        nc = lax.axis_size("c")
        wid = lax.axis_index("s") * nc + lax.axis_index("c")
        is_tail = wid == NUM_WORKERS - 1
        base = wid * b_per_w
        table_cp = pltpu.async_copy(scale_hbm, table_v, sem_t)

        @pl.when(~is_tail)
        def _():
            pltpu.async_copy(idx_hbm.at[pl.ds(base, b_per_w)],
                             idx_v.at[pl.ds(0, b_per_w)], sem_i).wait()

        @pl.when(is_tail)
        def _():
            pltpu.async_copy(idx_hbm.at[pl.ds(tail_base, tail_len)],
                             idx_v.at[pl.ds(0, tail_len)], sem_i).wait()

        table_cp.wait()

        def step(i, carry):
            off = pl.multiple_of(i * LANES, LANES)
            iv = idx_v[pl.ds(off, LANES)]
            iv = jnp.minimum(jnp.maximum(iv, 0), g - 1)
            v = plsc.load_gather(table_v, [iv])
            out_v[pl.ds(off, LANES)] = 1.0 / (v + EPS_DIV)
            return carry

        lax.fori_loop(0, b_per_w // LANES, step, 0)

        @pl.when(~is_tail)
        def _():
            pltpu.sync_copy(out_v.at[pl.ds(0, b_per_w)],
                            out_hbm.at[0, pl.ds(base, b_per_w)])

        @pl.when(is_tail)
        def _():
            pltpu.sync_copy(out_v.at[pl.ds(0, tail_out)],
                            out_hbm.at[0, pl.ds(tail_base, tail_out)])

    fn = pl.kernel(
        body,
        out_type=jax.ShapeDtypeStruct((1, g_pad), jnp.float32),
        mesh=mesh,
        compiler_params=pltpu.CompilerParams(needs_layout_passes=False),
        scratch_types=[
            pltpu.VMEM((g,), jnp.float32),
            pltpu.VMEM((b_per_w,), jnp.int32),
            pltpu.VMEM((b_per_w,), jnp.float32),
            pltpu.SemaphoreType.DMA,
            pltpu.SemaphoreType.DMA,
        ],
    )
    return fn(scale_g, idx)


def _tc_scale_mul(x_ng, recip_row, block_rows):
    n, g = x_ng.shape
    g_pad = recip_row.shape[1]

    def body(x_ref, r_ref, o_ref):
        o_ref[...] = x_ref[...] * r_ref[...][:, :g]

    return pl.pallas_call(
        body,
        grid=(pl.cdiv(n, block_rows),),
        in_specs=[
            pl.BlockSpec((block_rows, g), lambda i: (i, 0)),
            pl.BlockSpec((1, g_pad), lambda i: (0, 0)),
        ],
        out_specs=pl.BlockSpec((block_rows, g), lambda i: (i, 0)),
        out_shape=jax.ShapeDtypeStruct((n, g), jnp.float32),
        compiler_params=pltpu.CompilerParams(vmem_limit_bytes=128 * 1024 * 1024),
    )(x_ng, recip_row)


@jax.jit
def kernel(x_ng, scale_g, idx):
    n, g = x_ng.shape
    chunk = NUM_WORKERS * LANES
    g_pad = ((g + chunk - 1) // chunk) * chunk
    recip_row = _sc_recip_gather(scale_g, idx, g_pad)
    return _tc_scale_mul(x_ng, recip_row, block_rows=112)


# speedup vs baseline: 1.0367x; 1.0315x over previous
import jax
import jax.numpy as jnp
from jax import lax
from jax.experimental import pallas as pl
from jax.experimental.pallas import tpu as pltpu
from jax.experimental.pallas import tpu_sc as plsc

EPS_DIV = 1e-06
LANES = 16
NUM_WORKERS = 32


def _sc_recip_gather(scale_g, idx, g_pad):
    g = idx.shape[0]
    b_per_w = g_pad // NUM_WORKERS
    tail_base = (NUM_WORKERS - 1) * b_per_w
    tail_len = g - tail_base
    tail_out = ((tail_len + 127) // 128) * 128
    mesh = plsc.VectorSubcoreMesh(core_axis_name="c", subcore_axis_name="s")

    n_rows = b_per_w // 128
    tail_full_rows = tail_len // 128

    def body(scale_hbm, idx_hbm, out_hbm, idx_v, vals_v, out_v, sem_i, sem_g):
        nc = lax.axis_size("c")
        wid = lax.axis_index("s") * nc + lax.axis_index("c")
        is_tail = wid == NUM_WORKERS - 1
        base = wid * b_per_w

        @pl.when(~is_tail)
        def _():
            cps = [pltpu.async_copy(idx_hbm.at[pl.ds(base + j * 128, 128)],
                                    idx_v.at[j], sem_i)
                   for j in range(n_rows)]
            for cp in cps:
                cp.wait()

        @pl.when(is_tail)
        def _():
            rem = tail_len - tail_full_rows * 128
            cps = [pltpu.async_copy(idx_hbm.at[pl.ds(tail_base + j * 128, 128)],
                                    idx_v.at[j], sem_i)
                   for j in range(tail_full_rows)]
            cps.append(pltpu.async_copy(
                idx_hbm.at[pl.ds(tail_base + tail_full_rows * 128, rem)],
                idx_v.at[tail_full_rows, pl.ds(0, rem)], sem_i))
            for cp in cps:
                cp.wait()
            for j in range(tail_full_rows, n_rows):
                for v in range(128 // LANES):
                    off = v * LANES
                    iv = idx_v[j, pl.ds(off, LANES)]
                    idx_v[j, pl.ds(off, LANES)] = jnp.minimum(
                        jnp.maximum(iv, 0), g - 1)

        cps = [pltpu.async_copy(scale_hbm.at[idx_v.at[j]],
                                vals_v.at[pl.ds(j * 128, 128)], sem_g)
               for j in range(n_rows)]
        for cp in cps:
            cp.wait()

        def step(i, carry):
            off = pl.multiple_of(i * LANES, LANES)
            v = vals_v[pl.ds(off, LANES)]
            out_v[pl.ds(off, LANES)] = 1.0 / (v + EPS_DIV)
            return carry

        lax.fori_loop(0, b_per_w // LANES, step, 0)

        @pl.when(~is_tail)
        def _():
            pltpu.sync_copy(out_v.at[pl.ds(0, b_per_w)],
                            out_hbm.at[0, pl.ds(base, b_per_w)])

        @pl.when(is_tail)
        def _():
            pltpu.sync_copy(out_v.at[pl.ds(0, tail_out)],
                            out_hbm.at[0, pl.ds(tail_base, tail_out)])

    fn = pl.kernel(
        body,
        out_type=jax.ShapeDtypeStruct((1, g_pad), jnp.float32),
        mesh=mesh,
        compiler_params=pltpu.CompilerParams(needs_layout_passes=False),
        scratch_types=[
            pltpu.VMEM((n_rows, 128), jnp.int32),
            pltpu.VMEM((b_per_w,), jnp.float32),
            pltpu.VMEM((b_per_w,), jnp.float32),
            pltpu.SemaphoreType.DMA,
            pltpu.SemaphoreType.DMA,
        ],
    )
    return fn(scale_g, idx)


def _tc_scale_mul(x_ng, recip_row, block_rows):
    n, g = x_ng.shape
    g_pad = recip_row.shape[1]

    def body(x_ref, r_ref, o_ref):
        o_ref[...] = x_ref[...] * r_ref[...][:, :g]

    return pl.pallas_call(
        body,
        grid=(pl.cdiv(n, block_rows),),
        in_specs=[
            pl.BlockSpec((block_rows, g), lambda i: (i, 0)),
            pl.BlockSpec((1, g_pad), lambda i: (0, 0)),
        ],
        out_specs=pl.BlockSpec((block_rows, g), lambda i: (i, 0)),
        out_shape=jax.ShapeDtypeStruct((n, g), jnp.float32),
        compiler_params=pltpu.CompilerParams(vmem_limit_bytes=128 * 1024 * 1024),
    )(x_ng, recip_row)


@jax.jit
def kernel(x_ng, scale_g, idx):
    n, g = x_ng.shape
    chunk = NUM_WORKERS * LANES
    g_pad = ((g + chunk - 1) // chunk) * chunk
    recip_row = _sc_recip_gather(scale_g, idx, g_pad)
    return _tc_scale_mul(x_ng, recip_row, block_rows=112)


# speedup vs baseline: 1.0368x; 1.0001x over previous
import jax
import jax.numpy as jnp
from jax import lax
from jax.experimental import pallas as pl
from jax.experimental.pallas import tpu as pltpu
from jax.experimental.pallas import tpu_sc as plsc

EPS_DIV = 1e-06
LANES = 16
NUM_WORKERS = 32


def _sc_recip_gather(scale_g, idx, g_pad):
    g = idx.shape[0]
    b_per_w = g_pad // NUM_WORKERS
    tail_base = (NUM_WORKERS - 1) * b_per_w
    tail_len = g - tail_base
    tail_out = ((tail_len + 127) // 128) * 128
    mesh = plsc.VectorSubcoreMesh(core_axis_name="c", subcore_axis_name="s")

    n_rows = b_per_w // 128
    tail_full_rows = tail_len // 128

    def body(scale_hbm, idx_hbm, out_hbm, idx_v, vals_v, out_v, sem_i, sem_g):
        nc = lax.axis_size("c")
        wid = lax.axis_index("s") * nc + lax.axis_index("c")
        is_tail = wid == NUM_WORKERS - 1
        base = wid * b_per_w

        @pl.when(~is_tail)
        def _():
            cps = [pltpu.async_copy(idx_hbm.at[pl.ds(base + j * 128, 128)],
                                    idx_v.at[j], sem_i)
                   for j in range(n_rows)]
            for cp in cps:
                cp.wait()

        @pl.when(is_tail)
        def _():
            rem = tail_len - tail_full_rows * 128
            cps = [pltpu.async_copy(idx_hbm.at[pl.ds(tail_base + j * 128, 128)],
                                    idx_v.at[j], sem_i)
                   for j in range(tail_full_rows)]
            cps.append(pltpu.async_copy(
                idx_hbm.at[pl.ds(tail_base + tail_full_rows * 128, rem)],
                idx_v.at[tail_full_rows, pl.ds(0, rem)], sem_i))
            for cp in cps:
                cp.wait()
            for j in range(tail_full_rows, n_rows):
                for v in range(128 // LANES):
                    off = v * LANES
                    iv = idx_v[j, pl.ds(off, LANES)]
                    idx_v[j, pl.ds(off, LANES)] = jnp.minimum(
                        jnp.maximum(iv, 0), g - 1)

        cps = [pltpu.async_copy(scale_hbm.at[idx_v.at[j]],
                                vals_v.at[pl.ds(j * 128, 128)], sem_g)
               for j in range(n_rows)]
        for cp in cps:
            cp.wait()

        for i in range(b_per_w // LANES):
            off = i * LANES
            v = vals_v[pl.ds(off, LANES)]
            out_v[pl.ds(off, LANES)] = 1.0 / (v + EPS_DIV)

        @pl.when(~is_tail)
        def _():
            pltpu.sync_copy(out_v.at[pl.ds(0, b_per_w)],
                            out_hbm.at[0, pl.ds(base, b_per_w)])

        @pl.when(is_tail)
        def _():
            pltpu.sync_copy(out_v.at[pl.ds(0, tail_out)],
                            out_hbm.at[0, pl.ds(tail_base, tail_out)])

    fn = pl.kernel(
        body,
        out_type=jax.ShapeDtypeStruct((1, g_pad), jnp.float32),
        mesh=mesh,
        compiler_params=pltpu.CompilerParams(needs_layout_passes=False),
        scratch_types=[
            pltpu.VMEM((n_rows, 128), jnp.int32),
            pltpu.VMEM((b_per_w,), jnp.float32),
            pltpu.VMEM((b_per_w,), jnp.float32),
            pltpu.SemaphoreType.DMA,
            pltpu.SemaphoreType.DMA,
        ],
    )
    return fn(scale_g, idx)


def _tc_scale_mul(x_ng, recip_row, block_rows):
    n, g = x_ng.shape
    g_pad = recip_row.shape[1]

    def body(x_ref, r_ref, o_ref):
        o_ref[...] = x_ref[...] * r_ref[...][:, :g]

    return pl.pallas_call(
        body,
        grid=(pl.cdiv(n, block_rows),),
        in_specs=[
            pl.BlockSpec((block_rows, g), lambda i: (i, 0)),
            pl.BlockSpec((1, g_pad), lambda i: (0, 0)),
        ],
        out_specs=pl.BlockSpec((block_rows, g), lambda i: (i, 0)),
        out_shape=jax.ShapeDtypeStruct((n, g), jnp.float32),
        compiler_params=pltpu.CompilerParams(vmem_limit_bytes=128 * 1024 * 1024),
    )(x_ng, recip_row)


@jax.jit
def kernel(x_ng, scale_g, idx):
    n, g = x_ng.shape
    chunk = NUM_WORKERS * LANES
    g_pad = ((g + chunk - 1) // chunk) * chunk
    recip_row = _sc_recip_gather(scale_g, idx, g_pad)
    return _tc_scale_mul(x_ng, recip_row, block_rows=112)
